# Initial kernel scaffold; baseline (speedup 1.0000x reference)
import jax
import jax.numpy as jnp
from jax.experimental import pallas as pl


def kernel(atom_in_fea, nbr_fea, nbr_fea_idx1, nbr_fea_idx2, num_nbrs, crystal_atom_idx, We1, be1, We2, be2, We3, be3, Wv1, bv1, Wv2, bv2, Wv3, bv3, bn_gamma, bn_beta):
    raise NotImplementedError("write your pallas kernel here")


# speedup vs baseline: 4.7368x; 4.7368x over previous
import functools

import jax
import jax.numpy as jnp
from jax import lax
from jax.experimental import pallas as pl
from jax.experimental.pallas import tpu as pltpu
from jax.experimental.pallas import tpu_sc as plsc

M = 128
N_NODES = 10000
N_EDGES = 320000
N_CRYS = 256

NC = 2
NS = 16
NW = NC * NS
C = 80

F32 = jnp.float32


def _leaky(x):
    return jnp.where(x >= 0, x, 0.2 * x)


def _pq_body(atom_ref, w1at_ref, w1bt_ref, be1_ref, p_ref, q_ref):
    a = atom_ref[...]
    p_ref[...] = jnp.dot(a, w1at_ref[...], preferred_element_type=F32)
    q_ref[...] = jnp.dot(a, w1bt_ref[...], preferred_element_type=F32) + be1_ref[...]


def _pq(atom, w1at, w1bt, be1r):
    return pl.pallas_call(
        _pq_body,
        out_shape=[
            jax.ShapeDtypeStruct((N_NODES, M), F32),
            jax.ShapeDtypeStruct((N_NODES, M), F32),
        ],
    )(atom, w1at, w1bt, be1r)


_EPW = N_EDGES // NW
_GCHUNKS = _EPW // C


def _gather_body(p_hbm, q_hbm, i1_hbm, i2_hbm, out_hbm, i1_v, i2_v, a_v, b_v, sem):
    wid = lax.axis_index("s") * NC + lax.axis_index("c")
    base = wid * _EPW

    def chunk(j, carry):
        off = base + j * C
        pltpu.sync_copy(i1_hbm.at[pl.ds(off, C)], i1_v)
        pltpu.sync_copy(i2_hbm.at[pl.ds(off, C)], i2_v)
        cp1 = pltpu.async_copy(p_hbm.at[i1_v], a_v, sem)
        cp2 = pltpu.async_copy(q_hbm.at[i2_v], b_v, sem)
        cp1.wait()
        cp2.wait()

        def row(e, c2):
            for k in range(M // 16):
                s = pl.ds(k * 16, 16)
                a_v[e, s] = a_v[e, s] + b_v[e, s]
            return c2

        lax.fori_loop(0, C, row, 0)
        pltpu.sync_copy(a_v, out_hbm.at[pl.ds(off, C)])
        return carry

    lax.fori_loop(0, _GCHUNKS, chunk, 0)


def _gather(p, q, idx1, idx2):
    mesh = plsc.VectorSubcoreMesh(core_axis_name="c", subcore_axis_name="s")
    fn = functools.partial(
        pl.kernel,
        out_type=jax.ShapeDtypeStruct((N_EDGES, M), F32),
        mesh=mesh,
        scratch_types=[
            pltpu.VMEM((C,), jnp.int32),
            pltpu.VMEM((C,), jnp.int32),
            pltpu.VMEM((C, M), F32),
            pltpu.VMEM((C, M), F32),
            pltpu.SemaphoreType.DMA,
        ],
    )(_gather_body)
    return fn(p, q, idx1, idx2)


_BE = 3200


def _edge_body(g_ref, nbr_ref, w1ct_ref, w2t_ref, be2_ref, w3t_ref, be3_ref, ek_ref):
    nbr = nbr_ref[...]
    h = _leaky(g_ref[...] + jnp.dot(nbr, w1ct_ref[...], preferred_element_type=F32))
    h = _leaky(jnp.dot(h, w2t_ref[...], preferred_element_type=F32) + be2_ref[...])
    ek_ref[...] = nbr + jnp.dot(h, w3t_ref[...], preferred_element_type=F32) + be3_ref[...]


def _edge(g, nbr, w1ct, w2t, be2r, w3t, be3r):
    nblk = N_EDGES // _BE
    row_spec = pl.BlockSpec((_BE, M), lambda i: (i, 0))
    w_spec = pl.BlockSpec((M, M), lambda i: (0, 0))
    b_spec = pl.BlockSpec((1, M), lambda i: (0, 0))
    return pl.pallas_call(
        _edge_body,
        grid=(nblk,),
        in_specs=[row_spec, row_spec, w_spec, w_spec, b_spec, w_spec, b_spec],
        out_specs=row_spec,
        out_shape=jax.ShapeDtypeStruct((N_EDGES, M), F32),
        compiler_params=pltpu.CompilerParams(dimension_semantics=("arbitrary",)),
    )(g, nbr, w1ct, w2t, be2r, w3t, be3r)


_EPT = N_EDGES // NS
_SCHUNKS = _EPT // C
_NPAD = 10240
_RPT = _NPAD // NS
_ZR = 128


def _scatter_one(src_hbm, idx_hbm, out_hbm, idx_v, row_v, zero_v, acc_sh, sid):
    def zrow(e, c):
        for k in range(M // 16):
            zero_v[e, pl.ds(k * 16, 16)] = jnp.zeros((16,), F32)
        return c

    lax.fori_loop(0, _ZR, zrow, 0)
    for r in range(_RPT // _ZR):
        pltpu.sync_copy(zero_v, acc_sh.at[pl.ds(sid * _RPT + r * _ZR, _ZR)])
    plsc.subcore_barrier()

    base = sid * _EPT

    def chunk(j, carry):
        off = base + j * C
        pltpu.sync_copy(idx_hbm.at[pl.ds(off, C)], idx_v)
        pltpu.sync_copy(src_hbm.at[pl.ds(off, C)], row_v)
        pltpu.sync_copy(row_v, acc_sh.at[idx_v], add=True)
        return carry

    lax.fori_loop(0, _SCHUNKS, chunk, 0)
    plsc.subcore_barrier()
    pltpu.sync_copy(acc_sh.at[pl.ds(sid * _RPT, _RPT)],
                    out_hbm.at[pl.ds(sid * _RPT, _RPT)])


def _scatter_body(ek_hbm, nbr_hbm, i1_hbm, sek_hbm, snbr_hbm,
                  idx_v, row_v, zero_v, acc_sh):
    cid = lax.axis_index("c")
    sid = lax.axis_index("s")

    @pl.when(cid == 0)
    def _():
        _scatter_one(ek_hbm, i1_hbm, sek_hbm, idx_v, row_v, zero_v, acc_sh, sid)

    @pl.when(cid == 1)
    def _():
        _scatter_one(nbr_hbm, i1_hbm, snbr_hbm, idx_v, row_v, zero_v, acc_sh, sid)


def _scatter(ek, nbr, idx1):
    mesh = plsc.VectorSubcoreMesh(core_axis_name="c", subcore_axis_name="s")
    fn = functools.partial(
        pl.kernel,
        out_type=[
            jax.ShapeDtypeStruct((_NPAD, M), F32),
            jax.ShapeDtypeStruct((_NPAD, M), F32),
        ],
        mesh=mesh,
        scratch_types=[
            pltpu.VMEM((C,), jnp.int32),
            pltpu.VMEM((C, M), F32),
            pltpu.VMEM((_ZR, M), F32),
            pltpu.VMEM_SHARED((_NPAD, M), F32),
        ],
    )(_scatter_body)
    return fn(ek, nbr, idx1)


def _node_body(atom_ref, sek_ref, snbr_ref, num_ref, cidx_ref,
               wv1at_ref, wv1bt_ref, bv1_ref, wv2t_ref, bv2_ref,
               wv3t_ref, bv3_ref, gam_ref, bet_ref,
               vi_ref, anf_ref, gf_ref):
    atom = atom_ref[...]
    rnum = 1.0 / num_ref[...]
    eksum = sek_ref[...] * rnum
    rho = eksum - snbr_ref[...] * rnum
    g = _leaky(jnp.dot(atom, wv1at_ref[...], preferred_element_type=F32)
               + jnp.dot(rho, wv1bt_ref[...], preferred_element_type=F32)
               + bv1_ref[...])
    g = _leaky(jnp.dot(g, wv2t_ref[...], preferred_element_type=F32) + bv2_ref[...])
    vp = jnp.dot(g, wv3t_ref[...], preferred_element_type=F32) + bv3_ref[...]
    mean = jnp.mean(vp, axis=0, keepdims=True)
    var = jnp.mean((vp - mean) ** 2, axis=0, keepdims=True)
    vi = atom + (vp - mean) * lax.rsqrt(var + 1e-5) * gam_ref[...] + bet_ref[...]
    vi_ref[...] = vi
    anf = jnp.concatenate([vi, eksum], axis=1)
    anf_ref[...] = anf
    oht = (cidx_ref[...] ==
           lax.broadcasted_iota(jnp.int32, (N_CRYS, N_NODES), 0)).astype(F32)
    gf_ref[...] = jnp.dot(oht, anf, preferred_element_type=F32)


def _node(atom, sek, snbr, num_col, cidx_row,
          wv1at, wv1bt, bv1r, wv2t, bv2r, wv3t, bv3r, gamr, betr):
    return pl.pallas_call(
        _node_body,
        out_shape=[
            jax.ShapeDtypeStruct((N_NODES, M), F32),
            jax.ShapeDtypeStruct((N_NODES, 2 * M), F32),
            jax.ShapeDtypeStruct((N_CRYS, 2 * M), F32),
        ],
    )(atom, sek, snbr, num_col, cidx_row,
      wv1at, wv1bt, bv1r, wv2t, bv2r, wv3t, bv3r, gamr, betr)


def kernel(atom_in_fea, nbr_fea, nbr_fea_idx1, nbr_fea_idx2, num_nbrs,
           crystal_atom_idx, We1, be1, We2, be2, We3, be3,
           Wv1, bv1, Wv2, bv2, Wv3, bv3, bn_gamma, bn_beta):
    n, m = atom_in_fea.shape
    w1at = We1[:, :m].T
    w1bt = We1[:, m:2 * m].T
    w1ct = We1[:, 2 * m:].T
    w2t = We2.T
    w3t = We3.T
    wv1at = Wv1[:, :m].T
    wv1bt = Wv1[:, m:].T
    wv2t = Wv2.T
    wv3t = Wv3.T
    be1r = be1.reshape(1, m)
    be2r = be2.reshape(1, m)
    be3r = be3.reshape(1, m)
    bv1r = bv1.reshape(1, m)
    bv2r = bv2.reshape(1, m)
    bv3r = bv3.reshape(1, m)
    gamr = bn_gamma.reshape(1, m)
    betr = bn_beta.reshape(1, m)
    num_col = num_nbrs.reshape(n, 1)
    cidx_row = crystal_atom_idx.reshape(1, n)

    p, q = _pq(atom_in_fea, w1at, w1bt, be1r)
    g = _gather(p, q, nbr_fea_idx1, nbr_fea_idx2)
    ek = _edge(g, nbr_fea, w1ct, w2t, be2r, w3t, be3r)
    s_ek_pad, s_nbr_pad = _scatter(ek, nbr_fea, nbr_fea_idx1)
    s_ek = s_ek_pad[:n]
    s_nbr = s_nbr_pad[:n]
    vi, anf, gf = _node(atom_in_fea, s_ek, s_nbr, num_col, cidx_row,
                        wv1at, wv1bt, bv1r, wv2t, bv2r, wv3t, bv3r, gamr, betr)
    return (ek, vi, gf, anf)


# speedup vs baseline: 8.8520x; 1.8688x over previous
import functools

import jax
import jax.numpy as jnp
from jax import lax
from jax.experimental import pallas as pl
from jax.experimental.pallas import tpu as pltpu
from jax.experimental.pallas import tpu_sc as plsc

M = 128
N_NODES = 10000
N_EDGES = 320000
N_CRYS = 256

NC = 2
NS = 16
NW = NC * NS
C = 80

F32 = jnp.float32


def _leaky(x):
    return jnp.where(x >= 0, x, 0.2 * x)


def _pq_body(atom_ref, w1at_ref, w1bt_ref, be1_ref, p_ref, q_ref):
    a = atom_ref[...]
    p_ref[...] = jnp.dot(a, w1at_ref[...], preferred_element_type=F32)
    q_ref[...] = jnp.dot(a, w1bt_ref[...], preferred_element_type=F32) + be1_ref[...]


def _pq(atom, w1at, w1bt, be1r):
    return pl.pallas_call(
        _pq_body,
        out_shape=[
            jax.ShapeDtypeStruct((N_NODES, M), F32),
            jax.ShapeDtypeStruct((N_NODES, M), F32),
        ],
    )(atom, w1at, w1bt, be1r)


_EPW = N_EDGES // NW
_GCHUNKS = _EPW // C


def _rows_add(a, b, o):
    def row(e, carry):
        for k in range(M // 16):
            s = pl.ds(k * 16, 16)
            o[e, s] = a[e, s] + b[e, s]
        return carry

    lax.fori_loop(0, C, row, 0)


def _gather_body(p_hbm, q_hbm, i1_hbm, i2_hbm, out_hbm,
                 i1v, i2v, a0, a1, b0, b1, o0, o1, sg0, sg1, sw0, sw1):
    wid = lax.axis_index("s") * NC + lax.axis_index("c")
    ebase = wid * _EPW

    pltpu.sync_copy(i1_hbm.at[pl.ds(ebase, _EPW)], i1v)
    pltpu.sync_copy(i2_hbm.at[pl.ds(ebase, _EPW)], i2v)

    def issue_gather(c, a, b, sg):
        pltpu.async_copy(p_hbm.at[i1v.at[pl.ds(c * C, C)]], a, sg)
        pltpu.async_copy(q_hbm.at[i2v.at[pl.ds(c * C, C)]], b, sg)

    def wait_gather(sg, buf):
        pltpu.make_async_copy(p_hbm.at[pl.ds(0, C)], buf, sg).wait()
        pltpu.make_async_copy(p_hbm.at[pl.ds(0, C)], buf, sg).wait()

    def wait_write(o, sw):
        pltpu.make_async_copy(o, out_hbm.at[pl.ds(0, C)], sw).wait()

    issue_gather(0, a0, b0, sg0)

    def phase(c, aX, bX, oX, sgX, swX, aY, bY, sgY, have_w2):
        issue_gather(c + 1, aY, bY, sgY)
        wait_gather(sgX, aX)

        @pl.when(have_w2)
        def _():
            wait_write(oX, swX)

        _rows_add(aX, bX, oX)
        pltpu.async_copy(oX, out_hbm.at[pl.ds(ebase + c * C, C)], swX)

    def pair(j2, carry):
        c0 = 2 * j2
        phase(c0, a0, b0, o0, sg0, sw0, a1, b1, sg1, j2 >= 1)
        phase(c0 + 1, a1, b1, o1, sg1, sw1, a0, b0, sg0, j2 >= 1)
        return carry

    lax.fori_loop(0, (_GCHUNKS - 1) // 2, pair, 0)

    c_last = _GCHUNKS - 1
    wait_gather(sg0, a0)
    wait_write(o0, sw0)
    _rows_add(a0, b0, o0)
    pltpu.async_copy(o0, out_hbm.at[pl.ds(ebase + c_last * C, C)], sw0)
    wait_write(o1, sw1)
    wait_write(o0, sw0)


def _gather(p, q, idx1, idx2):
    mesh = plsc.VectorSubcoreMesh(core_axis_name="c", subcore_axis_name="s")
    fn = functools.partial(
        pl.kernel,
        out_type=jax.ShapeDtypeStruct((N_EDGES, M), F32),
        mesh=mesh,
        scratch_types=[
            pltpu.VMEM((_EPW,), jnp.int32),
            pltpu.VMEM((_EPW,), jnp.int32),
            pltpu.VMEM((C, M), F32),
            pltpu.VMEM((C, M), F32),
            pltpu.VMEM((C, M), F32),
            pltpu.VMEM((C, M), F32),
            pltpu.VMEM((C, M), F32),
            pltpu.VMEM((C, M), F32),
            pltpu.SemaphoreType.DMA,
            pltpu.SemaphoreType.DMA,
            pltpu.SemaphoreType.DMA,
            pltpu.SemaphoreType.DMA,
        ],
    )(_gather_body)
    return fn(p, q, idx1, idx2)


_BE = 3200


def _edge_body(g_ref, nbr_ref, w1ct_ref, w2t_ref, be2_ref, w3t_ref, be3_ref, ek_ref):
    nbr = nbr_ref[...]
    h = _leaky(g_ref[...] + jnp.dot(nbr, w1ct_ref[...], preferred_element_type=F32))
    h = _leaky(jnp.dot(h, w2t_ref[...], preferred_element_type=F32) + be2_ref[...])
    ek_ref[...] = nbr + jnp.dot(h, w3t_ref[...], preferred_element_type=F32) + be3_ref[...]


def _edge(g, nbr, w1ct, w2t, be2r, w3t, be3r):
    nblk = N_EDGES // _BE
    row_spec = pl.BlockSpec((_BE, M), lambda i: (i, 0))
    w_spec = pl.BlockSpec((M, M), lambda i: (0, 0))
    b_spec = pl.BlockSpec((1, M), lambda i: (0, 0))
    return pl.pallas_call(
        _edge_body,
        grid=(nblk,),
        in_specs=[row_spec, row_spec, w_spec, w_spec, b_spec, w_spec, b_spec],
        out_specs=row_spec,
        out_shape=jax.ShapeDtypeStruct((N_EDGES, M), F32),
        compiler_params=pltpu.CompilerParams(dimension_semantics=("arbitrary",)),
    )(g, nbr, w1ct, w2t, be2r, w3t, be3r)


_EPT = N_EDGES // NS
_SCHUNKS = _EPT // C
_NPAD = 10240
_RPT = _NPAD // NS
_ZR = 128


def _scatter_one(src_hbm, idx_hbm, out_hbm, i0, i1, r0, r1, zero_v, acc_sh,
                 sl0, sl1, ss0, ss1, sid):
    def zrow(e, c):
        for k in range(M // 16):
            zero_v[e, pl.ds(k * 16, 16)] = jnp.zeros((16,), F32)
        return c

    lax.fori_loop(0, _ZR, zrow, 0)
    for r in range(_RPT // _ZR):
        pltpu.sync_copy(zero_v, acc_sh.at[pl.ds(sid * _RPT + r * _ZR, _ZR)])
    plsc.subcore_barrier()

    base = sid * _EPT

    def issue_load(c, iv, rv, sl):
        pltpu.async_copy(idx_hbm.at[pl.ds(base + c * C, C)], iv, sl)
        pltpu.async_copy(src_hbm.at[pl.ds(base + c * C, C)], rv, sl)

    def wait_load(iv, rv, sl):
        pltpu.make_async_copy(idx_hbm.at[pl.ds(0, C)], iv, sl).wait()
        pltpu.make_async_copy(src_hbm.at[pl.ds(0, C)], rv, sl).wait()

    def wait_scat(rv, ss):
        pltpu.make_async_copy(src_hbm.at[pl.ds(0, C)], rv, ss).wait()

    issue_load(0, i0, r0, sl0)

    def phase(c, iX, rX, slX, ssX, iY, rY, slY, ssY, have_prev, have_next):
        if have_prev is True:
            wait_scat(rY, ssY)
        else:
            @pl.when(have_prev)
            def _():
                wait_scat(rY, ssY)

        if have_next is True:
            issue_load(c + 1, iY, rY, slY)
        else:
            @pl.when(have_next)
            def _():
                issue_load(c + 1, iY, rY, slY)

        wait_load(iX, rX, slX)
        pltpu.async_copy(rX, acc_sh.at[iX], ssX, add=True)

    def pair(j2, carry):
        c0 = 2 * j2
        phase(c0, i0, r0, sl0, ss0, i1, r1, sl1, ss1, j2 >= 1, True)
        phase(c0 + 1, i1, r1, sl1, ss1, i0, r0, sl0, ss0, True,
              j2 <= _SCHUNKS // 2 - 2)
        return carry

    lax.fori_loop(0, _SCHUNKS // 2, pair, 0)
    wait_scat(r1, ss1)
    plsc.subcore_barrier()
    pltpu.sync_copy(acc_sh.at[pl.ds(sid * _RPT, _RPT)],
                    out_hbm.at[pl.ds(sid * _RPT, _RPT)])


def _scatter_body(ek_hbm, nbr_hbm, i1_hbm, sek_hbm, snbr_hbm,
                  i0, i1, r0, r1, zero_v, acc_sh, sl0, sl1, ss0, ss1):
    cid = lax.axis_index("c")
    sid = lax.axis_index("s")

    @pl.when(cid == 0)
    def _():
        _scatter_one(ek_hbm, i1_hbm, sek_hbm, i0, i1, r0, r1, zero_v, acc_sh,
                     sl0, sl1, ss0, ss1, sid)

    @pl.when(cid == 1)
    def _():
        _scatter_one(nbr_hbm, i1_hbm, snbr_hbm, i0, i1, r0, r1, zero_v, acc_sh,
                     sl0, sl1, ss0, ss1, sid)


def _scatter(ek, nbr, idx1):
    mesh = plsc.VectorSubcoreMesh(core_axis_name="c", subcore_axis_name="s")
    fn = functools.partial(
        pl.kernel,
        out_type=[
            jax.ShapeDtypeStruct((_NPAD, M), F32),
            jax.ShapeDtypeStruct((_NPAD, M), F32),
        ],
        mesh=mesh,
        scratch_types=[
            pltpu.VMEM((C,), jnp.int32),
            pltpu.VMEM((C,), jnp.int32),
            pltpu.VMEM((C, M), F32),
            pltpu.VMEM((C, M), F32),
            pltpu.VMEM((_ZR, M), F32),
            pltpu.VMEM_SHARED((_NPAD, M), F32),
            pltpu.SemaphoreType.DMA,
            pltpu.SemaphoreType.DMA,
            pltpu.SemaphoreType.DMA,
            pltpu.SemaphoreType.DMA,
        ],
    )(_scatter_body)
    return fn(ek, nbr, idx1)


def _node_body(atom_ref, sek_ref, snbr_ref, num_ref, cidx_ref,
               wv1at_ref, wv1bt_ref, bv1_ref, wv2t_ref, bv2_ref,
               wv3t_ref, bv3_ref, gam_ref, bet_ref,
               vi_ref, anf_ref, gf_ref):
    atom = atom_ref[...]
    rnum = 1.0 / num_ref[...]
    eksum = sek_ref[...] * rnum
    rho = eksum - snbr_ref[...] * rnum
    g = _leaky(jnp.dot(atom, wv1at_ref[...], preferred_element_type=F32)
               + jnp.dot(rho, wv1bt_ref[...], preferred_element_type=F32)
               + bv1_ref[...])
    g = _leaky(jnp.dot(g, wv2t_ref[...], preferred_element_type=F32) + bv2_ref[...])
    vp = jnp.dot(g, wv3t_ref[...], preferred_element_type=F32) + bv3_ref[...]
    mean = jnp.mean(vp, axis=0, keepdims=True)
    var = jnp.mean((vp - mean) ** 2, axis=0, keepdims=True)
    vi = atom + (vp - mean) * lax.rsqrt(var + 1e-5) * gam_ref[...] + bet_ref[...]
    vi_ref[...] = vi
    anf = jnp.concatenate([vi, eksum], axis=1)
    anf_ref[...] = anf
    oht = (cidx_ref[...] ==
           lax.broadcasted_iota(jnp.int32, (N_CRYS, N_NODES), 0)).astype(F32)
    gf_ref[...] = jnp.dot(oht, anf, preferred_element_type=F32)


def _node(atom, sek, snbr, num_col, cidx_row,
          wv1at, wv1bt, bv1r, wv2t, bv2r, wv3t, bv3r, gamr, betr):
    return pl.pallas_call(
        _node_body,
        out_shape=[
            jax.ShapeDtypeStruct((N_NODES, M), F32),
            jax.ShapeDtypeStruct((N_NODES, 2 * M), F32),
            jax.ShapeDtypeStruct((N_CRYS, 2 * M), F32),
        ],
    )(atom, sek, snbr, num_col, cidx_row,
      wv1at, wv1bt, bv1r, wv2t, bv2r, wv3t, bv3r, gamr, betr)


def kernel(atom_in_fea, nbr_fea, nbr_fea_idx1, nbr_fea_idx2, num_nbrs,
           crystal_atom_idx, We1, be1, We2, be2, We3, be3,
           Wv1, bv1, Wv2, bv2, Wv3, bv3, bn_gamma, bn_beta):
    n, m = atom_in_fea.shape
    w1at = We1[:, :m].T
    w1bt = We1[:, m:2 * m].T
    w1ct = We1[:, 2 * m:].T
    w2t = We2.T
    w3t = We3.T
    wv1at = Wv1[:, :m].T
    wv1bt = Wv1[:, m:].T
    wv2t = Wv2.T
    wv3t = Wv3.T
    be1r = be1.reshape(1, m)
    be2r = be2.reshape(1, m)
    be3r = be3.reshape(1, m)
    bv1r = bv1.reshape(1, m)
    bv2r = bv2.reshape(1, m)
    bv3r = bv3.reshape(1, m)
    gamr = bn_gamma.reshape(1, m)
    betr = bn_beta.reshape(1, m)
    num_col = num_nbrs.reshape(n, 1)
    cidx_row = crystal_atom_idx.reshape(1, n)

    p, q = _pq(atom_in_fea, w1at, w1bt, be1r)
    g = _gather(p, q, nbr_fea_idx1, nbr_fea_idx2)
    ek = _edge(g, nbr_fea, w1ct, w2t, be2r, w3t, be3r)
    s_ek_pad, s_nbr_pad = _scatter(ek, nbr_fea, nbr_fea_idx1)
    s_ek = s_ek_pad[:n]
    s_nbr = s_nbr_pad[:n]
    vi, anf, gf = _node(atom_in_fea, s_ek, s_nbr, num_col, cidx_row,
                        wv1at, wv1bt, bv1r, wv2t, bv2r, wv3t, bv3r, gamr, betr)
    return (ek, vi, gf, anf)
